# baseline (speedup 1.0000x reference)
import functools

import jax
import jax.numpy as jnp
from jax import lax
from jax.experimental import pallas as pl
from jax.experimental.pallas import tpu as pltpu
from jax.experimental.pallas import tpu_sc as plsc

_N = 1_000_000
_C = 100
_NBINS = 15
_BK = 65536
_L = 16
_NW = 32
_THRESH = tuple(float(b) / _NBINS for b in range(_NBINS))

_BK2 = 16384
_N1 = 11 * _BK
_N2 = 16 * _BK2
_NV3 = _N - _N1 - _N2
_NP3 = -(-_NV3 // 512) * 512
_CHUNKS = (
    (_BK, 0, _N1 // _BK, _N1, _N1),
    (_BK2, _N1, _N2 // _BK2, _N2, _N2),
    (_BK2, _N1 + _N2, -(-_NV3 // _BK2), _NV3, _NP3),
)


def _make_stage1(bk, n_valid):
    def _stage1(pt_ref, labels_ref, conf_ref, d_ref):
        p = pt_ref[...]
        conf = jnp.max(p, axis=0)
        row = lax.broadcasted_iota(jnp.int32, (_C, bk), 0)
        pred = jnp.min(jnp.where(p == conf[None, :], row, _C), axis=0)
        acc = (pred == labels_ref[...]).astype(jnp.float32)
        gidx = pl.program_id(0) * bk + lax.broadcasted_iota(
            jnp.int32, (bk,), 0
        )
        valid = gidx < n_valid
        conf_ref[...] = jnp.where(valid, conf, 0.0)
        d_ref[...] = jnp.where(valid, conf - acc, 0.0)

    return _stage1


def _run_stage1(pt, labels1, bk, off_rows, grid, n_valid, n_pad):
    off_blocks = off_rows // bk
    return pl.pallas_call(
        _make_stage1(bk, n_valid),
        grid=(grid,),
        in_specs=[
            pl.BlockSpec((_C, bk), lambda i: (0, i + off_blocks)),
            pl.BlockSpec((bk,), lambda i: (i + off_blocks,)),
        ],
        out_specs=[
            pl.BlockSpec((bk,), lambda i: (i,)),
            pl.BlockSpec((bk,), lambda i: (i,)),
        ],
        out_shape=[
            jax.ShapeDtypeStruct((n_pad,), jnp.float32),
            jax.ShapeDtypeStruct((n_pad,), jnp.float32),
        ],
    )(pt, labels1)


@functools.cache
def _make_stage2(e):
    mesh = plsc.VectorSubcoreMesh(
        core_axis_name="c", subcore_axis_name="s", num_cores=2, num_subcores=16
    )

    @functools.partial(
        pl.kernel,
        out_type=jax.ShapeDtypeStruct((_NW, _NBINS, _L), jnp.float32),
        mesh=mesh,
        scratch_types=[
            pltpu.VMEM((e,), jnp.float32),
            pltpu.VMEM((e,), jnp.float32),
            pltpu.VMEM((_NBINS, _L), jnp.float32),
        ],
    )
    def _stage2(conf_hbm, d_hbm, out_hbm, conf_v, d_v, acc_v):
        wid = lax.axis_index("s") * 2 + lax.axis_index("c")
        base = wid * e
        pltpu.sync_copy(conf_hbm.at[pl.ds(base, e)], conf_v)
        pltpu.sync_copy(d_hbm.at[pl.ds(base, e)], d_v)

        zero = jnp.zeros((_L,), jnp.float32)

        def body(i, us):
            cv = conf_v[pl.ds(i * _L, _L)]
            dv = d_v[pl.ds(i * _L, _L)]
            return tuple(
                u + jnp.where(cv > t, dv, 0.0) for u, t in zip(us, _THRESH)
            )

        us = lax.fori_loop(0, e // _L, body, (zero,) * _NBINS)
        for b in range(_NBINS):
            nxt = us[b + 1] if b + 1 < _NBINS else zero
            acc_v[b, :] = us[b] - nxt
        pltpu.sync_copy(acc_v, out_hbm.at[wid])

    return _stage2


def _stage3(*refs):
    out_ref = refs[-1]
    x = refs[0][...]
    for r in refs[1:-1]:
        x = x + r[...]
    s = jnp.sum(jnp.sum(x, axis=0), axis=1)
    ece = jnp.sum(jnp.abs(s)) * (1.0 / _N)
    out_ref[...] = jnp.reshape(ece, (1, 1))


def kernel(probs, labels):
    pt = probs.T
    labels1 = labels.astype(jnp.int32)

    parts = []
    for bk, off, grid, n_valid, n_pad in _CHUNKS:
        conf, d = _run_stage1(pt, labels1, bk, off, grid, n_valid, n_pad)
        parts.append(_make_stage2(n_pad // _NW)(conf, d))

    ece = pl.pallas_call(
        _stage3,
        out_shape=jax.ShapeDtypeStruct((1, 1), jnp.float32),
    )(*parts)
    return ece.reshape(1)


# speedup vs baseline: 1.0177x; 1.0177x over previous
import functools

import jax
import jax.numpy as jnp
from jax import lax
from jax.experimental import pallas as pl
from jax.experimental.pallas import tpu as pltpu
from jax.experimental.pallas import tpu_sc as plsc

_N = 1_000_000
_C = 100
_NBINS = 15
_BK = 65536
_L = 16
_NW = 32
_THRESH = tuple(float(b) / _NBINS for b in range(_NBINS))

_N1 = 14 * _BK
_NV2 = _N - _N1
_NP2 = -(-_NV2 // 512) * 512
_CHUNKS = (
    (_BK, 0, _N1 // _BK, _N1, _N1),
    (_BK, _N1, -(-_NV2 // _BK), _NV2, _NP2),
)


def _make_stage1(bk, n_valid):
    def _stage1(pt_ref, labels_ref, conf_ref, d_ref):
        p = pt_ref[...]
        conf = jnp.max(p, axis=0)
        row = lax.broadcasted_iota(jnp.int32, (_C, bk), 0)
        pred = jnp.min(jnp.where(p == conf[None, :], row, _C), axis=0)
        acc = (pred == labels_ref[...]).astype(jnp.float32)
        gidx = pl.program_id(0) * bk + lax.broadcasted_iota(
            jnp.int32, (bk,), 0
        )
        valid = gidx < n_valid
        conf_ref[...] = jnp.where(valid, conf, 0.0)
        d_ref[...] = jnp.where(valid, conf - acc, 0.0)

    return _stage1


def _run_stage1(pt, labels1, bk, off_rows, grid, n_valid, n_pad):
    off_blocks = off_rows // bk
    return pl.pallas_call(
        _make_stage1(bk, n_valid),
        grid=(grid,),
        in_specs=[
            pl.BlockSpec((_C, bk), lambda i: (0, i + off_blocks)),
            pl.BlockSpec((bk,), lambda i: (i + off_blocks,)),
        ],
        out_specs=[
            pl.BlockSpec((bk,), lambda i: (i,)),
            pl.BlockSpec((bk,), lambda i: (i,)),
        ],
        out_shape=[
            jax.ShapeDtypeStruct((n_pad,), jnp.float32),
            jax.ShapeDtypeStruct((n_pad,), jnp.float32),
        ],
    )(pt, labels1)


@functools.cache
def _make_stage2(e):
    mesh = plsc.VectorSubcoreMesh(
        core_axis_name="c", subcore_axis_name="s", num_cores=2, num_subcores=16
    )

    @functools.partial(
        pl.kernel,
        out_type=jax.ShapeDtypeStruct((_NW, _NBINS, _L), jnp.float32),
        mesh=mesh,
        scratch_types=[
            pltpu.VMEM((e,), jnp.float32),
            pltpu.VMEM((e,), jnp.float32),
            pltpu.VMEM((_NBINS, _L), jnp.float32),
        ],
    )
    def _stage2(conf_hbm, d_hbm, out_hbm, conf_v, d_v, acc_v):
        wid = lax.axis_index("s") * 2 + lax.axis_index("c")
        base = wid * e
        pltpu.sync_copy(conf_hbm.at[pl.ds(base, e)], conf_v)
        pltpu.sync_copy(d_hbm.at[pl.ds(base, e)], d_v)

        zero = jnp.zeros((_L,), jnp.float32)

        def body(i, us):
            cv = conf_v[pl.ds(i * _L, _L)]
            dv = d_v[pl.ds(i * _L, _L)]
            return tuple(
                u + jnp.where(cv > t, dv, 0.0) for u, t in zip(us, _THRESH)
            )

        us = lax.fori_loop(0, e // _L, body, (zero,) * _NBINS)
        for b in range(_NBINS):
            nxt = us[b + 1] if b + 1 < _NBINS else zero
            acc_v[b, :] = us[b] - nxt
        pltpu.sync_copy(acc_v, out_hbm.at[wid])

    return _stage2


def _stage3(*refs):
    out_ref = refs[-1]
    x = refs[0][...]
    for r in refs[1:-1]:
        x = x + r[...]
    s = jnp.sum(jnp.sum(x, axis=0), axis=1)
    ece = jnp.sum(jnp.abs(s)) * (1.0 / _N)
    out_ref[...] = jnp.reshape(ece, (1, 1))


def kernel(probs, labels):
    pt = probs.T
    labels1 = labels.astype(jnp.int32)

    parts = []
    for bk, off, grid, n_valid, n_pad in _CHUNKS:
        conf, d = _run_stage1(pt, labels1, bk, off, grid, n_valid, n_pad)
        parts.append(_make_stage2(n_pad // _NW)(conf, d))

    ece = pl.pallas_call(
        _stage3,
        out_shape=jax.ShapeDtypeStruct((1, 1), jnp.float32),
    )(*parts)
    return ece.reshape(1)
